# baseline (speedup 1.0000x reference)
import functools

import numpy as np
import jax
import jax.numpy as jnp
from jax import lax
from jax.experimental import pallas as pl
from jax.experimental.pallas import tpu as pltpu
from jax.experimental.pallas import tpu_sc as plsc

_C_IN, _C_OUT, _R_OUT, _P, _T = 4, 64, 16, 5, 16
_B = _P * _T
_N = 1024
_M = _N * _B
_K = _C_IN * _B
_G = _K * _R_OUT
_KC = _C_OUT * _R_OUT

_NC, _NS = 2, 16
_NW = _NC * _NS
_ROWS_PER_W = _G // _NW
_CHUNK = 80
_NCHUNK = _ROWS_PER_W // _CHUNK

_BM = 4096


def _build_gather_idx() -> np.ndarray:
    b = np.arange(_B)
    ti, pi = b // _P, b % _P
    idx = np.empty((_B, _C_IN, _R_OUT), dtype=np.int32)
    for c in range(_C_IN):
        for j in range(_R_OUT):
            idx[:, c, j] = c * _B + ((ti + j) % _T) * _P + pi
    return idx.reshape(_NW, _NCHUNK, _CHUNK)


_IDX = _build_gather_idx()


def _sc_rotation_gather(weights, idx):
    mesh = plsc.VectorSubcoreMesh(core_axis_name="c", subcore_axis_name="s")

    @functools.partial(
        pl.kernel,
        mesh=mesh,
        out_type=jax.ShapeDtypeStruct((_G, _C_OUT), jnp.float32),
        scratch_types=[
            pltpu.VMEM((_NCHUNK, _CHUNK), jnp.int32),
            pltpu.VMEM((_NCHUNK, _CHUNK, _C_OUT), jnp.float32),
            pltpu.SemaphoreType.DMA,
        ],
        compiler_params=pltpu.CompilerParams(use_tc_tiling_on_sc=False),
    )
    def gather_kernel(w_hbm, idx_hbm, out_hbm, idx_v, rows_v, sem):
        wid = lax.axis_index("s") * _NC + lax.axis_index("c")
        pltpu.sync_copy(idx_hbm.at[wid], idx_v)
        copies = [
            pltpu.async_copy(w_hbm.at[idx_v.at[ch]], rows_v.at[ch], sem)
            for ch in range(_NCHUNK)
        ]
        for c in copies:
            c.wait()
        base = wid * _ROWS_PER_W
        for ch in range(_NCHUNK):
            pltpu.sync_copy(rows_v.at[ch], out_hbm.at[pl.ds(base + ch * _CHUNK, _CHUNK)])

    return gather_kernel(weights, idx)


def _conn_matvec_body(conn_ref, x_ref, h_ref):
    h_ref[...] = jnp.dot(conn_ref[...], x_ref[...], preferred_element_type=jnp.float32)


def _conn_matvec_tc(conn, x):
    return pl.pallas_call(
        _conn_matvec_body,
        grid=(_M // _BM,),
        in_specs=[
            pl.BlockSpec((_BM, _N), lambda i: (i, 0)),
            pl.BlockSpec((_N, _C_IN), lambda i: (0, 0)),
        ],
        out_specs=pl.BlockSpec((_BM, _C_IN), lambda i: (i, 0)),
        out_shape=jax.ShapeDtypeStruct((_M, _C_IN), jnp.float32),
        compiler_params=pltpu.CompilerParams(
            dimension_semantics=("arbitrary",),
            vmem_limit_bytes=100 * 1024 * 1024,
        ),
    )(conn, x)


def _mix_body(hr_ref, cc_ref, o_ref):
    p = lax.broadcasted_iota(jnp.int32, (_KC, _KC), 0)
    q = lax.broadcasted_iota(jnp.int32, (_KC, _KC), 1)
    s = jnp.where(q == (p % _C_OUT) * _R_OUT + p // _C_OUT, 1.0, 0.0)
    lw = jnp.dot(cc_ref[...], s, preferred_element_type=jnp.float32)
    o_ref[...] = jnp.dot(hr_ref[...], lw, preferred_element_type=jnp.float32)


def _mix(hr, cc):
    return pl.pallas_call(
        _mix_body,
        out_shape=jax.ShapeDtypeStruct((_N, _KC), jnp.float32),
    )(hr, cc)


def kernel(x, conn, weights):
    cg = _sc_rotation_gather(weights, jnp.asarray(_IDX))
    cc = cg.reshape(_K, _R_OUT * _C_OUT)
    h = _conn_matvec_tc(conn, x)
    hr = h.reshape(_N, _K)
    return _mix(hr, cc)


# speedup vs baseline: 1.0571x; 1.0571x over previous
import functools

import numpy as np
import jax
import jax.numpy as jnp
from jax import lax
from jax.experimental import pallas as pl
from jax.experimental.pallas import tpu as pltpu
from jax.experimental.pallas import tpu_sc as plsc

_C_IN, _C_OUT, _R_OUT, _P, _T = 4, 64, 16, 5, 16
_B = _P * _T
_N = 1024
_M = _N * _B
_K = _C_IN * _B
_G = _K * _R_OUT
_KC = _C_OUT * _R_OUT

_NC, _NS = 2, 16
_NW = _NC * _NS
_ROWS_PER_W = _G // _NW
_CHUNK = 80
_NCHUNK = _ROWS_PER_W // _CHUNK

_BM = 4096


def _build_gather_idx() -> np.ndarray:
    b = np.arange(_B)
    ti, pi = b // _P, b % _P
    idx = np.empty((_B, _C_IN, _R_OUT), dtype=np.int32)
    for c in range(_C_IN):
        for j in range(_R_OUT):
            idx[:, c, j] = c * _B + ((ti + j) % _T) * _P + pi
    return idx.reshape(_NW, _NCHUNK, _CHUNK)


_IDX = _build_gather_idx()


def _sc_rotation_gather(weights, idx):
    mesh = plsc.VectorSubcoreMesh(core_axis_name="c", subcore_axis_name="s")

    @functools.partial(
        pl.kernel,
        mesh=mesh,
        out_type=jax.ShapeDtypeStruct((_G, _C_OUT), jnp.float32),
        scratch_types=[
            pltpu.VMEM((_NCHUNK, _CHUNK), jnp.int32),
            pltpu.VMEM((_NCHUNK, _CHUNK, _C_OUT), jnp.float32),
            pltpu.SemaphoreType.DMA,
        ],
        compiler_params=pltpu.CompilerParams(use_tc_tiling_on_sc=False),
    )
    def gather_kernel(w_hbm, idx_hbm, out_hbm, idx_v, rows_v, sem):
        wid = lax.axis_index("s") * _NC + lax.axis_index("c")
        pltpu.sync_copy(idx_hbm.at[wid], idx_v)
        copies = [
            pltpu.async_copy(w_hbm.at[idx_v.at[ch]], rows_v.at[ch], sem)
            for ch in range(_NCHUNK)
        ]
        for c in copies:
            c.wait()
        base = wid * _ROWS_PER_W
        for ch in range(_NCHUNK):
            pltpu.sync_copy(rows_v.at[ch], out_hbm.at[pl.ds(base + ch * _CHUNK, _CHUNK)])

    return gather_kernel(weights, idx)


def _conn_matvec_body(conn_ref, x_ref, h_ref):
    h_ref[...] = jnp.dot(
        conn_ref[...], x_ref[...], preferred_element_type=jnp.float32
    ).astype(jnp.bfloat16)


def _conn_matvec_tc(conn, x):
    return pl.pallas_call(
        _conn_matvec_body,
        grid=(_M // _BM,),
        in_specs=[
            pl.BlockSpec((_BM, _N), lambda i: (i, 0)),
            pl.BlockSpec((_N, _C_IN), lambda i: (0, 0)),
        ],
        out_specs=pl.BlockSpec((_BM, _C_IN), lambda i: (i, 0)),
        out_shape=jax.ShapeDtypeStruct((_M, _C_IN), jnp.bfloat16),
        compiler_params=pltpu.CompilerParams(
            dimension_semantics=("arbitrary",),
            vmem_limit_bytes=100 * 1024 * 1024,
        ),
    )(conn, x)


def _mix_body(hr_ref, cc_ref, o_ref):
    p = lax.broadcasted_iota(jnp.int32, (_KC, _KC), 0)
    q = lax.broadcasted_iota(jnp.int32, (_KC, _KC), 1)
    s = jnp.where(q == (p % _C_OUT) * _R_OUT + p // _C_OUT, 1.0, 0.0)
    lw = jnp.dot(cc_ref[...], s, preferred_element_type=jnp.float32)
    o_ref[...] = jnp.dot(
        hr_ref[...], lw.astype(jnp.bfloat16), preferred_element_type=jnp.float32
    )


def _mix(hr, cc):
    return pl.pallas_call(
        _mix_body,
        out_shape=jax.ShapeDtypeStruct((_N, _KC), jnp.float32),
    )(hr, cc)


def kernel(x, conn, weights):
    cg = _sc_rotation_gather(weights, jnp.asarray(_IDX))
    cc = cg.reshape(_K, _R_OUT * _C_OUT)
    h = _conn_matvec_tc(conn, x)
    hr = h.reshape(_N, _K)
    return _mix(hr, cc)


# speedup vs baseline: 1.0656x; 1.0080x over previous
import functools

import numpy as np
import jax
import jax.numpy as jnp
from jax import lax
from jax.experimental import pallas as pl
from jax.experimental.pallas import tpu as pltpu
from jax.experimental.pallas import tpu_sc as plsc

_C_IN, _C_OUT, _R_OUT, _P, _T = 4, 64, 16, 5, 16
_B = _P * _T
_N = 1024
_M = _N * _B
_K = _C_IN * _B
_G = _K * _R_OUT
_KC = _C_OUT * _R_OUT

_NC, _NS = 2, 16
_NW = _NC * _NS
_ROWS_PER_W = _G // _NW
_CHUNK = 80
_NCHUNK = _ROWS_PER_W // _CHUNK

_BM = 4096


def _build_gather_idx() -> np.ndarray:
    b = np.arange(_B)
    ti, pi = b // _P, b % _P
    idx = np.empty((_B, _C_IN, _R_OUT), dtype=np.int32)
    for c in range(_C_IN):
        for j in range(_R_OUT):
            idx[:, c, j] = c * _B + ((ti + j) % _T) * _P + pi
    return idx.reshape(_NW, _NCHUNK, _CHUNK)


_IDX = _build_gather_idx()


def _sc_rotation_gather(weights, idx):
    mesh = plsc.VectorSubcoreMesh(core_axis_name="c", subcore_axis_name="s")

    @functools.partial(
        pl.kernel,
        mesh=mesh,
        out_type=jax.ShapeDtypeStruct((_G, _C_OUT), jnp.float32),
        scratch_types=[
            pltpu.VMEM((_NCHUNK, _CHUNK), jnp.int32),
            pltpu.VMEM((_NCHUNK, _CHUNK, _C_OUT), jnp.float32),
            pltpu.SemaphoreType.DMA,
        ],
        compiler_params=pltpu.CompilerParams(use_tc_tiling_on_sc=False),
    )
    def gather_kernel(w_hbm, idx_hbm, out_hbm, idx_v, rows_v, sem):
        wid = lax.axis_index("s") * _NC + lax.axis_index("c")
        pltpu.sync_copy(idx_hbm.at[wid], idx_v)
        copies = [
            pltpu.async_copy(w_hbm.at[idx_v.at[ch]], rows_v.at[ch], sem)
            for ch in range(_NCHUNK)
        ]
        for c in copies:
            c.wait()
        base = wid * _ROWS_PER_W
        for ch in range(_NCHUNK):
            pltpu.sync_copy(rows_v.at[ch], out_hbm.at[pl.ds(base + ch * _CHUNK, _CHUNK)])

    return gather_kernel(weights, idx)


def _conn_matvec_body(conn_ref, x_ref, h_ref):
    h_ref[...] = jnp.dot(
        conn_ref[...], x_ref[...], preferred_element_type=jnp.float32
    ).astype(jnp.bfloat16)


def _conn_matvec_tc(conn, x):
    return pl.pallas_call(
        _conn_matvec_body,
        grid=(_M // _BM,),
        in_specs=[
            pl.BlockSpec((_BM, _N), lambda i: (i, 0)),
            pl.BlockSpec((_N, _C_IN), lambda i: (0, 0)),
        ],
        out_specs=pl.BlockSpec((_BM, _C_IN), lambda i: (i, 0)),
        out_shape=jax.ShapeDtypeStruct((_M, _C_IN), jnp.bfloat16),
        compiler_params=pltpu.CompilerParams(
            dimension_semantics=("arbitrary",),
            vmem_limit_bytes=100 * 1024 * 1024,
        ),
    )(conn, x)


def _mix_body(hr_ref, cc_ref, o_ref):
    p = lax.broadcasted_iota(jnp.int32, (_KC, _KC), 0)
    q = lax.broadcasted_iota(jnp.int32, (_KC, _KC), 1)
    s = jnp.where(q == (p % _C_OUT) * _R_OUT + p // _C_OUT, 1.0, 0.0).astype(
        jnp.bfloat16
    )
    lw = jnp.dot(
        cc_ref[...].astype(jnp.bfloat16), s, preferred_element_type=jnp.float32
    ).astype(jnp.bfloat16)
    o_ref[...] = jnp.dot(hr_ref[...], lw, preferred_element_type=jnp.float32)


def _mix(hr, cc):
    return pl.pallas_call(
        _mix_body,
        out_shape=jax.ShapeDtypeStruct((_N, _KC), jnp.float32),
    )(hr, cc)


def kernel(x, conn, weights):
    cg = _sc_rotation_gather(weights, jnp.asarray(_IDX))
    cc = cg.reshape(_K, _R_OUT * _C_OUT)
    h = _conn_matvec_tc(conn, x)
    hr = h.reshape(_N, _K)
    return _mix(hr, cc)
